# baseline (speedup 1.0000x reference)
import functools

import jax
import jax.numpy as jnp
import numpy as np
from jax.experimental import pallas as pl
from jax.experimental.pallas import tpu as pltpu
from jax.sharding import Mesh, PartitionSpec as P

try:
    from jax import shard_map as _shard_map
except ImportError:
    from jax.experimental.shard_map import shard_map as _shard_map


def _h_body(feat_ref, w_ref, h_ref):
    h_ref[...] = jax.lax.dot_general(
        feat_ref[...], w_ref[...],
        dimension_numbers=(((1,), (1,)), ((), ())),
        preferred_element_type=jnp.float32,
    )


def _gcn_body(a_ref, adj_ref, h_ref, bias_ref, out_ref):
    out = jax.lax.dot_general(
        adj_ref[...], h_ref[...],
        dimension_numbers=(((1,), (0,)), ((), ())),
        preferred_element_type=jnp.float32,
    )
    out = out + bias_ref[...]
    alpha = a_ref[0, 0]
    out_ref[...] = jnp.where(out >= 0, out, alpha * out)


def _gcn(feat2, adj2, W, bias2, a2, bm):
    rows, n = adj2.shape
    d_out = W.shape[0]

    h = pl.pallas_call(
        _h_body,
        out_shape=jax.ShapeDtypeStruct((n, d_out), jnp.float32),
    )(feat2, W)

    return pl.pallas_call(
        _gcn_body,
        grid=(rows // bm,),
        in_specs=[
            pl.BlockSpec(memory_space=pltpu.SMEM),
            pl.BlockSpec((bm, n), lambda i: (i, 0)),
            pl.BlockSpec((n, d_out), lambda i: (0, 0)),
            pl.BlockSpec((1, d_out), lambda i: (0, 0)),
        ],
        out_specs=pl.BlockSpec((bm, d_out), lambda i: (i, 0)),
        out_shape=jax.ShapeDtypeStruct((rows, d_out), jnp.float32),
        compiler_params=pltpu.CompilerParams(
            dimension_semantics=("parallel",),
        ),
    )(a2, adj2, h, bias2)


def kernel(feat, adj, W, bias, prelu_a):
    b, n, d_in = feat.shape
    d_out = W.shape[0]
    feat2 = feat.reshape(n, d_in)
    adj2 = adj.reshape(n, n)
    bias2 = bias.reshape(1, d_out)
    a2 = jnp.asarray(prelu_a, jnp.float32).reshape(1, 1)
    bm = 256 if n % 256 == 0 else n

    devs = jax.devices()
    ndev = len(devs)
    if ndev > 1 and n % (ndev * bm) == 0:
        mesh = Mesh(np.array(devs), ("x",))
        gcn = _shard_map(
            functools.partial(_gcn, bm=bm),
            mesh=mesh,
            in_specs=(P(), P("x"), P(), P(), P()),
            out_specs=P("x"),
            check_vma=False,
        )
        out = gcn(feat2, adj2, W, bias2, a2)
    else:
        out = _gcn(feat2, adj2, W, bias2, a2, bm)
    return out.reshape(b, n, d_out)


# speedup vs baseline: 3.9208x; 3.9208x over previous
import jax
import jax.numpy as jnp
from jax.experimental import pallas as pl
from jax.experimental.pallas import tpu as pltpu


def _h_body(feat_ref, w_ref, h_ref):
    h_ref[...] = jax.lax.dot_general(
        feat_ref[...], w_ref[...],
        dimension_numbers=(((1,), (1,)), ((), ())),
        preferred_element_type=jnp.float32,
    )


def _gcn_body(a_ref, adj_ref, h_ref, bias_ref, out_ref):
    out = jax.lax.dot_general(
        adj_ref[...], h_ref[...],
        dimension_numbers=(((1,), (0,)), ((), ())),
        preferred_element_type=jnp.float32,
    )
    out = out + bias_ref[...]
    alpha = a_ref[0, 0]
    out_ref[...] = jnp.where(out >= 0, out, alpha * out)


def _gcn(feat2, adj2, W, bias2, a2, bm):
    rows, n = adj2.shape
    d_out = W.shape[0]

    h = pl.pallas_call(
        _h_body,
        out_shape=jax.ShapeDtypeStruct((n, d_out), jnp.float32),
    )(feat2, W)

    return pl.pallas_call(
        _gcn_body,
        grid=(rows // bm,),
        in_specs=[
            pl.BlockSpec(memory_space=pltpu.SMEM),
            pl.BlockSpec((bm, n), lambda i: (i, 0)),
            pl.BlockSpec((n, d_out), lambda i: (0, 0)),
            pl.BlockSpec((1, d_out), lambda i: (0, 0)),
        ],
        out_specs=pl.BlockSpec((bm, d_out), lambda i: (i, 0)),
        out_shape=jax.ShapeDtypeStruct((rows, d_out), jnp.float32),
        compiler_params=pltpu.CompilerParams(
            dimension_semantics=("parallel",),
        ),
    )(a2, adj2, h, bias2)


def kernel(feat, adj, W, bias, prelu_a):
    b, n, d_in = feat.shape
    d_out = W.shape[0]
    feat2 = feat.reshape(n, d_in)
    adj2 = adj.reshape(n, n)
    bias2 = bias.reshape(1, d_out)
    a2 = jnp.asarray(prelu_a, jnp.float32).reshape(1, 1)
    bm = 128 if n % 128 == 0 else n

    out = _gcn(feat2, adj2, W, bias2, a2, bm)
    return out.reshape(b, n, d_out)


# speedup vs baseline: 3.9428x; 1.0056x over previous
import jax
import jax.numpy as jnp
from jax.experimental import pallas as pl
from jax.experimental.pallas import tpu as pltpu


def _h_body(feat_ref, w_ref, h_ref):
    h_ref[...] = jax.lax.dot_general(
        feat_ref[...], w_ref[...],
        dimension_numbers=(((1,), (1,)), ((), ())),
        preferred_element_type=jnp.float32,
    )


def _gcn_body(a_ref, adj_ref, h_ref, bias_ref, out_ref):
    out = jax.lax.dot_general(
        adj_ref[...], h_ref[...],
        dimension_numbers=(((1,), (0,)), ((), ())),
        preferred_element_type=jnp.float32,
    )
    out = out + bias_ref[...]
    alpha = a_ref[0, 0]
    out_ref[...] = jnp.where(out >= 0, out, alpha * out)


def _gcn(feat2, adj2, W, bias2, a2, bm):
    rows, n = adj2.shape
    d_out = W.shape[0]

    h = pl.pallas_call(
        _h_body,
        out_shape=jax.ShapeDtypeStruct((n, d_out), jnp.float32),
    )(feat2, W)

    return pl.pallas_call(
        _gcn_body,
        grid=(rows // bm,),
        in_specs=[
            pl.BlockSpec(memory_space=pltpu.SMEM),
            pl.BlockSpec((bm, n), lambda i: (i, 0)),
            pl.BlockSpec((n, d_out), lambda i: (0, 0)),
            pl.BlockSpec((1, d_out), lambda i: (0, 0)),
        ],
        out_specs=pl.BlockSpec((bm, d_out), lambda i: (i, 0)),
        out_shape=jax.ShapeDtypeStruct((rows, d_out), jnp.float32),
        compiler_params=pltpu.CompilerParams(
            dimension_semantics=("parallel",),
            vmem_limit_bytes=100 * 1024 * 1024,
        ),
    )(a2, adj2, h, bias2)


def kernel(feat, adj, W, bias, prelu_a):
    b, n, d_in = feat.shape
    d_out = W.shape[0]
    feat2 = feat.reshape(n, d_in)
    adj2 = adj.reshape(n, n)
    bias2 = bias.reshape(1, d_out)
    a2 = jnp.asarray(prelu_a, jnp.float32).reshape(1, 1)
    bm = 256 if n % 256 == 0 else n

    out = _gcn(feat2, adj2, W, bias2, a2, bm)
    return out.reshape(b, n, d_out)


# speedup vs baseline: 4.0429x; 1.0254x over previous
import functools

import jax
import jax.numpy as jnp
from jax.experimental import pallas as pl
from jax.experimental.pallas import tpu as pltpu


def _gcn_body(a_ref, feat_ref, w_ref, adj_ref, bias_ref, out_ref, h_ref):
    @pl.when(pl.program_id(0) == 0)
    def _():
        h_ref[...] = jax.lax.dot_general(
            feat_ref[...], w_ref[...],
            dimension_numbers=(((1,), (1,)), ((), ())),
            preferred_element_type=jnp.float32,
        )

    out = jax.lax.dot_general(
        adj_ref[...], h_ref[...],
        dimension_numbers=(((1,), (0,)), ((), ())),
        preferred_element_type=jnp.float32,
    )
    out = out + bias_ref[...]
    alpha = a_ref[0, 0]
    out_ref[...] = jnp.where(out >= 0, out, alpha * out)


@functools.partial(jax.jit, static_argnames=("bm",))
def _gcn(feat2, adj2, W, bias2, a2, bm):
    n, d_in = feat2.shape
    d_out = W.shape[0]

    return pl.pallas_call(
        _gcn_body,
        grid=(n // bm,),
        in_specs=[
            pl.BlockSpec(memory_space=pltpu.SMEM),
            pl.BlockSpec((n, d_in), lambda i: (0, 0)),
            pl.BlockSpec((d_out, d_in), lambda i: (0, 0)),
            pl.BlockSpec((bm, n), lambda i: (i, 0)),
            pl.BlockSpec((1, d_out), lambda i: (0, 0)),
        ],
        out_specs=pl.BlockSpec((bm, d_out), lambda i: (i, 0)),
        out_shape=jax.ShapeDtypeStruct((n, d_out), jnp.float32),
        scratch_shapes=[pltpu.VMEM((n, d_out), jnp.float32)],
        compiler_params=pltpu.CompilerParams(
            dimension_semantics=("arbitrary",),
        ),
    )(a2, feat2, W, adj2, bias2)


def kernel(feat, adj, W, bias, prelu_a):
    b, n, d_in = feat.shape
    d_out = W.shape[0]
    feat2 = feat.reshape(n, d_in)
    adj2 = adj.reshape(n, n)
    bias2 = bias.reshape(1, d_out)
    a2 = jnp.asarray(prelu_a, jnp.float32).reshape(1, 1)
    bm = 256 if n % 256 == 0 else n
    out = _gcn(feat2, adj2, W, bias2, a2, bm)
    return out.reshape(b, n, d_out)
